# Initial kernel scaffold; baseline (speedup 1.0000x reference)
import jax
import jax.numpy as jnp
from jax.experimental import pallas as pl


def kernel(x):
    raise NotImplementedError("write your pallas kernel here")


# speedup vs baseline: 1.7405x; 1.7405x over previous
import functools

import jax
import jax.numpy as jnp
from jax import lax
from jax.experimental import pallas as pl
from jax.experimental.pallas import tpu as pltpu
from jax.experimental.pallas import tpu_sc as plsc

B, N, S, D = 4, 32, 64, 128
ROW = S * D
T = 6 + (N - 6) * 36
NC, NS = 2, 16
NW = NC * NS
NI = N - 6
ITEMS = B * NI


def _body(x_hbm, out_hbm, shared):
    cid = lax.axis_index("c")
    sid = lax.axis_index("s")
    wid = sid * NC + cid

    rps = (B * N) // NS
    pltpu.sync_copy(
        x_hbm.at[pl.ds(sid * rps, rps)],
        shared.at[pl.ds(sid * rps, rps)],
    )
    plsc.subcore_barrier()

    @pl.when(wid < B)
    def _():
        pltpu.sync_copy(
            shared.at[pl.ds(wid * N + 26, 6)],
            out_hbm.at[pl.ds(wid * T, 6)],
        )

    for g in range(6):
        j0 = (wid + 8 * g) % NW

        def do_item(item):
            bi = item // NI
            i = item % NI
            src0 = bi * N
            dst0 = bi * T + 6 + 36 * i + 6 * g
            if g > 0:
                pltpu.sync_copy(
                    shared.at[pl.ds(src0 + 26, g)],
                    out_hbm.at[pl.ds(dst0, g)],
                )
            pltpu.sync_copy(
                shared.at[pl.ds(src0 + i, 1)],
                out_hbm.at[pl.ds(dst0 + g, 1)],
            )
            if g < 5:
                pltpu.sync_copy(
                    shared.at[pl.ds(src0 + 27 + g, 5 - g)],
                    out_hbm.at[pl.ds(dst0 + g + 1, 5 - g)],
                )

        for k in range(3):
            do_item(j0 + NW * k)

        @pl.when(j0 < ITEMS - 3 * NW)
        def _():
            do_item(j0 + NW * 3)


@functools.partial(
    pl.kernel,
    out_type=jax.ShapeDtypeStruct((B * T, S, D), jnp.float32),
    mesh=plsc.VectorSubcoreMesh(core_axis_name="c", subcore_axis_name="s"),
    scratch_types=[pltpu.VMEM_SHARED((B * N, S, D), jnp.float32)],
)
def _gather_rows(x_hbm, out_hbm, shared):
    _body(x_hbm, out_hbm, shared)


def kernel(x):
    b, n, s, d = x.shape
    out = _gather_rows(x.reshape(b * n, s, d))
    return out.reshape(b, 6, T // 6, s, d)


# speedup vs baseline: 1.9242x; 1.1056x over previous
import functools

import jax
import jax.numpy as jnp
from jax import lax
from jax.experimental import pallas as pl
from jax.experimental.pallas import tpu as pltpu
from jax.experimental.pallas import tpu_sc as plsc

B, N, S, D = 4, 32, 64, 128
T = 6 + (N - 6) * 36
NC, NS = 2, 16
NW = NC * NS
NI = N - 6
ITEMS = B * NI
MAXPEND = 8


def _body(x_hbm, out_hbm, shared, sem):
    cid = lax.axis_index("c")
    sid = lax.axis_index("s")
    wid = sid * NC + cid

    rps = (B * N) // NS
    pltpu.sync_copy(
        x_hbm.at[pl.ds(sid * rps, rps)],
        shared.at[pl.ds(sid * rps, rps)],
    )
    plsc.subcore_barrier()

    pending = []

    def fire(src, dst):
        while len(pending) >= MAXPEND:
            pending.pop(0).wait()
        pending.append(pltpu.async_copy(src, dst, sem))

    @pl.when(wid < B)
    def _():
        pltpu.sync_copy(
            shared.at[pl.ds(wid * N + 26, 6)],
            out_hbm.at[pl.ds(wid * T, 6)],
        )

    for g in range(6):
        j0 = (wid + 8 * g) % NW

        def do_item(item, copy):
            bi = item // NI
            i = item % NI
            src0 = bi * N
            dst0 = bi * T + 6 + 36 * i + 6 * g
            if g > 0:
                copy(shared.at[pl.ds(src0 + 26, g)],
                     out_hbm.at[pl.ds(dst0, g)])
            copy(shared.at[pl.ds(src0 + i, 1)],
                 out_hbm.at[pl.ds(dst0 + g, 1)])
            if g < 5:
                copy(shared.at[pl.ds(src0 + 27 + g, 5 - g)],
                     out_hbm.at[pl.ds(dst0 + g + 1, 5 - g)])

        for k in range(3):
            do_item(j0 + NW * k, fire)

        @pl.when(j0 < ITEMS - 3 * NW)
        def _():
            do_item(j0 + NW * 3, lambda s_, d_: pltpu.sync_copy(s_, d_))

    for d in pending:
        d.wait()


@functools.partial(
    pl.kernel,
    out_type=jax.ShapeDtypeStruct((B * T, S, D), jnp.float32),
    mesh=plsc.VectorSubcoreMesh(core_axis_name="c", subcore_axis_name="s"),
    scratch_types=[
        pltpu.VMEM_SHARED((B * N, S, D), jnp.float32),
        pltpu.SemaphoreType.DMA,
    ],
)
def _gather_rows(x_hbm, out_hbm, shared, sem):
    _body(x_hbm, out_hbm, shared, sem)


def kernel(x):
    b, n, s, d = x.shape
    out = _gather_rows(x.reshape(b * n, s, d))
    return out.reshape(b, 6, T // 6, s, d)


# speedup vs baseline: 1.9831x; 1.0306x over previous
import functools

import jax
import jax.numpy as jnp
from jax import lax
from jax.experimental import pallas as pl
from jax.experimental.pallas import tpu as pltpu
from jax.experimental.pallas import tpu_sc as plsc

B, N, S, D = 4, 32, 64, 128
T = 6 + (N - 6) * 36
NC, NS = 2, 16
NW = NC * NS
NI = N - 6
ITEMS = B * NI
MAXPEND = 16


def _body(x_hbm, out_hbm, shared, sem):
    cid = lax.axis_index("c")
    sid = lax.axis_index("s")
    wid = sid * NC + cid

    rps = (B * N) // NS
    pltpu.sync_copy(
        x_hbm.at[pl.ds(sid * rps, rps)],
        shared.at[pl.ds(sid * rps, rps)],
    )
    plsc.subcore_barrier()

    pending = []

    def fire(src, dst):
        while len(pending) >= MAXPEND:
            pending.pop(0).wait()
        pending.append(pltpu.async_copy(src, dst, sem))

    @pl.when(wid < B)
    def _():
        pltpu.sync_copy(
            shared.at[pl.ds(wid * N + 26, 6)],
            out_hbm.at[pl.ds(wid * T, 6)],
        )

    for g in range(6):
        j0 = (wid + 8 * g) % NW

        def do_item(item, copy):
            bi = item // NI
            i = item % NI
            src0 = bi * N
            dst0 = bi * T + 6 + 36 * i + 6 * g
            if g > 0:
                copy(shared.at[pl.ds(src0 + 26, g)],
                     out_hbm.at[pl.ds(dst0, g)])
            copy(shared.at[pl.ds(src0 + i, 1)],
                 out_hbm.at[pl.ds(dst0 + g, 1)])
            if g < 5:
                copy(shared.at[pl.ds(src0 + 27 + g, 5 - g)],
                     out_hbm.at[pl.ds(dst0 + g + 1, 5 - g)])

        for k in range(3):
            do_item(j0 + NW * k, fire)

        @pl.when(j0 < ITEMS - 3 * NW)
        def _():
            local = []
            do_item(j0 + NW * 3,
                    lambda s_, d_: local.append(pltpu.async_copy(s_, d_, sem)))
            for d_ in local:
                d_.wait()

    for d in pending:
        d.wait()


@functools.partial(
    pl.kernel,
    out_type=jax.ShapeDtypeStruct((B * T, S, D), jnp.float32),
    mesh=plsc.VectorSubcoreMesh(core_axis_name="c", subcore_axis_name="s"),
    scratch_types=[
        pltpu.VMEM_SHARED((B * N, S, D), jnp.float32),
        pltpu.SemaphoreType.DMA,
    ],
)
def _gather_rows(x_hbm, out_hbm, shared, sem):
    _body(x_hbm, out_hbm, shared, sem)


def kernel(x):
    b, n, s, d = x.shape
    out = _gather_rows(x.reshape(b * n, s, d))
    return out.reshape(b, 6, T // 6, s, d)
